# Initial kernel scaffold; baseline (speedup 1.0000x reference)
import jax
import jax.numpy as jnp
from jax.experimental import pallas as pl


def kernel(x, edge_index, edge_attr, candidate, e1, n1a, n1b, e2, n2a, n2b, e3, n3a, n3b, mfin):
    raise NotImplementedError("write your pallas kernel here")


# speedup vs baseline: 1.3638x; 1.3638x over previous
import functools

import jax
import jax.numpy as jnp
from jax import lax
from jax.experimental import pallas as pl
from jax.experimental.pallas import tpu as pltpu
from jax.experimental.pallas import tpu_sc as plsc

F32 = jnp.float32
NC, NS, LANES = 2, 16, 16
NW = NC * NS
CBG = 128
CBS = 128
BE = 512
BN = 512

@functools.cache
def _mesh():
    return plsc.VectorSubcoreMesh(core_axis_name="c", subcore_axis_name="s",
                                  num_cores=NC, num_subcores=NS)


def _ln(h, lw, lb, eps=1e-5):
    m = jnp.mean(h, axis=-1, keepdims=True)
    v = jnp.mean((h - m) ** 2, axis=-1, keepdims=True)
    return (h - m) * lax.rsqrt(v + eps) * lw + lb


def _tables_body(x_ref, wr_ref, wc_ref, wv_ref, b1_ref, c1_ref, pr_ref, k_ref):
    x = x_ref[...]
    pr_ref[...] = jnp.dot(x, wr_ref[...], preferred_element_type=F32) + b1_ref[...]
    kc = jnp.dot(x, wc_ref[...], preferred_element_type=F32)
    kq = jnp.dot(x, wv_ref[...], preferred_element_type=F32) + c1_ref[...]
    k_ref[...] = jnp.concatenate([kc, kq], axis=1)


def _tc_tables(x, wr, wc, wv, b1, c1):
    np_, dx = x.shape
    h = wr.shape[1]
    grid = (np_ // BN,)
    return pl.pallas_call(
        _tables_body,
        grid=grid,
        in_specs=[
            pl.BlockSpec((BN, dx), lambda i: (i, 0)),
            pl.BlockSpec((dx, h), lambda i: (0, 0)),
            pl.BlockSpec((dx, h), lambda i: (0, 0)),
            pl.BlockSpec((dx, h), lambda i: (0, 0)),
            pl.BlockSpec((1, h), lambda i: (0, 0)),
            pl.BlockSpec((1, h), lambda i: (0, 0)),
        ],
        out_specs=[
            pl.BlockSpec((BN, h), lambda i: (i, 0)),
            pl.BlockSpec((BN, 2 * h), lambda i: (i, 0)),
        ],
        out_shape=[
            jax.ShapeDtypeStruct((np_, h), F32),
            jax.ShapeDtypeStruct((np_, 2 * h), F32),
        ],
    )(x, wr, wc, wv, b1, c1)


def _sc_gather(pr, ktab, rowg, colg):
    ep = rowg.shape[0]
    h = pr.shape[1]
    epw = ep // NW
    nch = epw // CBG

    @functools.partial(
        pl.kernel,
        out_type=[jax.ShapeDtypeStruct((ep, h), F32),
                  jax.ShapeDtypeStruct((ep, 2 * h), F32)],
        mesh=_mesh(),
        scratch_types=[
            pltpu.VMEM((CBG,), jnp.int32),
            pltpu.VMEM((CBG,), jnp.int32),
            pltpu.VMEM((CBG, h), F32),
            pltpu.VMEM((CBG, 2 * h), F32),
            pltpu.SemaphoreType.DMA,
            pltpu.SemaphoreType.DMA,
        ],
    )
    def k(pr_hbm, k_hbm, row_hbm, col_hbm, gr_hbm, gk_hbm,
          idxr, idxc, bufr, bufk, sem1, sem2):
        wid = lax.axis_index("s") * NC + lax.axis_index("c")
        base = wid * epw

        def body(j, carry):
            e0 = base + j * CBG
            pltpu.sync_copy(row_hbm.at[pl.ds(e0, CBG)], idxr)
            pltpu.sync_copy(col_hbm.at[pl.ds(e0, CBG)], idxc)
            cp1 = pltpu.async_copy(pr_hbm.at[idxr], bufr, sem1)
            cp2 = pltpu.async_copy(k_hbm.at[idxc], bufk, sem2)
            cp1.wait()
            pltpu.sync_copy(bufr, gr_hbm.at[pl.ds(e0, CBG)])
            cp2.wait()
            pltpu.sync_copy(bufk, gk_hbm.at[pl.ds(e0, CBG)])
            return carry

        lax.fori_loop(0, nch, body, 0)

    return k(pr, ktab, rowg, colg)


def _edge_body(gr_ref, gk_ref, ein_ref, we_ref, w2_ref, b2_ref, lwe_ref,
               lbe_ref, ve_ref, v2_ref, c2_ref, lwn_ref, lbn_ref,
               ea_ref, out_ref):
    h = gr_ref.shape[1]
    pre = (gr_ref[...] + gk_ref[:, :h]
           + jnp.dot(ein_ref[...], we_ref[...], preferred_element_type=F32))
    t = _ln(jnp.maximum(pre, 0.0), lwe_ref[...], lbe_ref[...])
    ea = jnp.dot(t, w2_ref[...], preferred_element_type=F32) + b2_ref[...]
    ea_ref[...] = ea
    pre2 = gk_ref[:, h:] + jnp.dot(ea, ve_ref[...], preferred_element_type=F32)
    u = _ln(jnp.maximum(pre2, 0.0), lwn_ref[...], lbn_ref[...])
    out_ref[...] = jnp.dot(u, v2_ref[...], preferred_element_type=F32) + c2_ref[...]


def _tc_edge(gr, gk, ein, we, w2, b2, lwe, lbe, ve, v2, c2, lwn, lbn):
    ep, h = gr.shape
    de = ein.shape[1]
    grid = (ep // BE,)
    wspec = lambda s: pl.BlockSpec(s, lambda i: (0, 0))
    return pl.pallas_call(
        _edge_body,
        grid=grid,
        in_specs=[
            pl.BlockSpec((BE, h), lambda i: (i, 0)),
            pl.BlockSpec((BE, 2 * h), lambda i: (i, 0)),
            pl.BlockSpec((BE, de), lambda i: (i, 0)),
            wspec((de, h)), wspec((h, h)), wspec((1, h)), wspec((1, h)),
            wspec((1, h)), wspec((h, h)), wspec((h, h)), wspec((1, h)),
            wspec((1, h)), wspec((1, h)),
        ],
        out_specs=[
            pl.BlockSpec((BE, h), lambda i: (i, 0)),
            pl.BlockSpec((BE, h), lambda i: (i, 0)),
        ],
        out_shape=[
            jax.ShapeDtypeStruct((ep, h), F32),
            jax.ShapeDtypeStruct((ep, h), F32),
        ],
    )(gr, gk, ein, we, w2, b2, lwe, lbe, ve, v2, c2, lwn, lbn)


def _sc_scatter(out_e, rows_scat, zeros_h):
    ep = rows_scat.shape[0]
    h = out_e.shape[1]
    hh = h // NC
    eps = ep // NS
    nch = eps // CBS
    np_ = zeros_h.shape[0]
    nps = np_ // NS

    @functools.partial(
        pl.kernel,
        out_type=jax.ShapeDtypeStruct((np_, h), F32),
        mesh=_mesh(),
        scratch_types=[
            pltpu.VMEM((CBS,), jnp.int32),
            pltpu.VMEM((CBS, hh), F32),
            pltpu.VMEM_SHARED((np_, hh), F32),
        ],
    )
    def k(out_hbm, row_hbm, z_hbm, sums_hbm, idxv, datv, acc):
        c = lax.axis_index("c")
        s = lax.axis_index("s")
        pltpu.sync_copy(z_hbm.at[pl.ds(s * nps, nps)],
                        acc.at[pl.ds(s * nps, nps)])
        plsc.subcore_barrier()

        def run(col0):
            def body(j, carry):
                e0 = s * eps + j * CBS
                pltpu.sync_copy(row_hbm.at[pl.ds(e0, CBS)], idxv)
                pltpu.sync_copy(out_hbm.at[pl.ds(e0, CBS), pl.ds(col0, hh)], datv)
                pltpu.sync_copy(datv, acc.at[idxv], add=True)
                return carry
            lax.fori_loop(0, nch, body, 0)

        @pl.when(c == 0)
        def _():
            run(0)

        @pl.when(c == 1)
        def _():
            run(hh)

        plsc.subcore_barrier()

        @pl.when(c == 0)
        def _():
            pltpu.sync_copy(acc.at[pl.ds(s * nps, nps)],
                            sums_hbm.at[pl.ds(s * nps, nps), pl.ds(0, hh)])

        @pl.when(c == 1)
        def _():
            pltpu.sync_copy(acc.at[pl.ds(s * nps, nps)],
                            sums_hbm.at[pl.ds(s * nps, nps), pl.ds(hh, hh)])

    return k(out_e, rows_scat, zeros_h)


def _sc_counts(rows_scat, ones_hbm, zeros_h):
    ep = rows_scat.shape[0]
    epw = ep // NW
    nch = epw // CBS
    np_, cw = zeros_h.shape
    nps = np_ // NS

    @functools.partial(
        pl.kernel,
        out_type=jax.ShapeDtypeStruct((NC * np_, cw), F32),
        mesh=_mesh(),
        scratch_types=[
            pltpu.VMEM((CBS,), jnp.int32),
            pltpu.VMEM((CBS, cw), F32),
            pltpu.VMEM_SHARED((np_, cw), F32),
        ],
    )
    def k(row_hbm, ones_ref, z_hbm, cnt_hbm, idxv, onev, acc):
        c = lax.axis_index("c")
        s = lax.axis_index("s")
        wid = s * NC + c
        pltpu.sync_copy(ones_ref, onev)
        pltpu.sync_copy(z_hbm.at[pl.ds(s * nps, nps)],
                        acc.at[pl.ds(s * nps, nps)])
        plsc.subcore_barrier()

        def body(j, carry):
            e0 = wid * epw + j * CBS
            pltpu.sync_copy(row_hbm.at[pl.ds(e0, CBS)], idxv)
            pltpu.sync_copy(onev, acc.at[idxv], add=True)
            return carry

        lax.fori_loop(0, nch, body, 0)
        plsc.subcore_barrier()

        @pl.when(c == 0)
        def _():
            pltpu.sync_copy(acc.at[pl.ds(s * nps, nps)],
                            cnt_hbm.at[pl.ds(s * nps, nps)])

        @pl.when(c == 1)
        def _():
            pltpu.sync_copy(acc.at[pl.ds(s * nps, nps)],
                            cnt_hbm.at[pl.ds(np_ + s * nps, nps)])

    return k(rows_scat, ones_hbm, zeros_h)


def _node_body(x_ref, sums_ref, c0_ref, c1_ref, ua_ref, ub_ref, d1_ref,
               lw_ref, lb_ref, u2_ref, d2_ref, xo_ref):
    cnt = c0_ref[:, 0:1] + c1_ref[:, 0:1]
    inv = 1.0 / jnp.maximum(cnt, 1.0)
    agg = sums_ref[...] * inv
    pre = (jnp.dot(x_ref[...], ua_ref[...], preferred_element_type=F32)
           + jnp.dot(agg, ub_ref[...], preferred_element_type=F32) + d1_ref[...])
    hh = _ln(jnp.maximum(pre, 0.0), lw_ref[...], lb_ref[...])
    xo_ref[...] = jnp.dot(hh, u2_ref[...], preferred_element_type=F32) + d2_ref[...]


def _tc_node(x, sums, c0, c1, ua, ub, d1, lw, lb, u2, d2):
    np_, dx = x.shape
    h = sums.shape[1]
    grid = (np_ // BN,)
    wspec = lambda s: pl.BlockSpec(s, lambda i: (0, 0))
    return pl.pallas_call(
        _node_body,
        grid=grid,
        in_specs=[
            pl.BlockSpec((BN, dx), lambda i: (i, 0)),
            pl.BlockSpec((BN, h), lambda i: (i, 0)),
            pl.BlockSpec((BN, c0.shape[1]), lambda i: (i, 0)),
            pl.BlockSpec((BN, c1.shape[1]), lambda i: (i, 0)),
            wspec((dx, h)), wspec((h, h)), wspec((1, h)), wspec((1, h)),
            wspec((1, h)), wspec((h, h)), wspec((1, h)),
        ],
        out_specs=pl.BlockSpec((BN, h), lambda i: (i, 0)),
        out_shape=jax.ShapeDtypeStruct((np_, h), F32),
    )(x, sums, c0, c1, ua, ub, d1, lw, lb, u2, d2)


def _sc_parents(x, cand16):
    h = x.shape[1]

    @functools.partial(
        pl.kernel,
        out_type=jax.ShapeDtypeStruct((LANES, h), F32),
        mesh=_mesh(),
        scratch_types=[
            pltpu.VMEM((LANES,), jnp.int32),
            pltpu.VMEM((LANES, h), F32),
            pltpu.SemaphoreType.DMA,
        ],
    )
    def k(x_hbm, cand_hbm, par_hbm, idxv, bufv, sem):
        c = lax.axis_index("c")
        s = lax.axis_index("s")

        @pl.when(jnp.logical_and(c == 0, s == 0))
        def _():
            pltpu.sync_copy(cand_hbm, idxv)
            pltpu.async_copy(x_hbm.at[idxv], bufv, sem).wait()
            pltpu.sync_copy(bufv, par_hbm)

    return k(x, cand16)


def _final_body(par_ref, w1_ref, b1_ref, w2_ref, b2_ref, lw_ref, lb_ref,
                o_ref, *, n_par):
    h = par_ref.shape[1]
    p = par_ref[...]
    acc = jnp.zeros((1, h), F32)
    for j in range(n_par):
        acc = acc + jnp.dot(p[j:j + 1, :], w1_ref[j * h:(j + 1) * h, :],
                            preferred_element_type=F32)
    hid = jnp.maximum(acc + b1_ref[...], 0.0)
    o = jnp.dot(hid, w2_ref[...], preferred_element_type=F32) + b2_ref[...]
    o = _ln(o, lw_ref[...], lb_ref[...])
    o = 1.0 / (1.0 + jnp.exp(-o))
    o_ref[...] = jnp.broadcast_to(o, o_ref.shape)


def _tc_final(parents, w1, b1, w2, b2, lw, lb, n_par):
    h = parents.shape[1]
    mo = w2.shape[1]
    return pl.pallas_call(
        functools.partial(_final_body, n_par=n_par),
        out_shape=jax.ShapeDtypeStruct((8, mo), F32),
    )(parents, w1, b1, w2, b2, lw, lb)


def _pad_rows(a, n):
    return jnp.pad(a, ((0, n - a.shape[0]),) + ((0, 0),) * (a.ndim - 1))


def kernel(x, edge_index, edge_attr, candidate, e1, n1a, n1b, e2, n2a, n2b,
           e3, n3a, n3b, mfin):
    n, df = x.shape
    e = edge_index.shape[1]
    h = e1[0].shape[1]
    s_par = candidate.shape[0] - 1

    np_ = ((n + BN - 1) // BN) * BN
    align = NW * CBG
    ep = ((e + align - 1) // align) * align

    row = edge_index[0].astype(jnp.int32)
    col = edge_index[1].astype(jnp.int32)
    rowg = jnp.pad(row, (0, ep - e))
    colg = jnp.pad(col, (0, ep - e))
    rows_scat = jnp.pad(row, (0, ep - e), constant_values=np_ - 1)
    ein = _pad_rows(edge_attr.astype(F32), ep)
    xc = _pad_rows(x.astype(F32), np_)
    zeros_h = jnp.zeros((np_, h // NC), F32)
    ones_blk = jnp.ones((CBS, h // NC), F32)
    cand16 = jnp.zeros((LANES,), jnp.int32).at[:s_par].set(
        candidate[1:1 + s_par].astype(jnp.int32))

    cnt = _sc_counts(rows_scat, ones_blk, zeros_h)
    c0, c1cnt = cnt[:np_], cnt[np_:]

    for pe, pn1, pn2 in ((e1, n1a, n1b), (e2, n2a, n2b), (e3, n3a, n3b)):
        w1, b1, lwe, lbe, w2, b2 = pe
        v1, cb1, lwn, lbn, v2, c2 = pn1
        u1, d1, lwm, lbm, u2, d2 = pn2
        dx = xc.shape[1]
        wr, wc, we = w1[:dx], w1[dx:2 * dx], w1[2 * dx:]
        vx, ve = v1[:dx], v1[dx:]
        ua, ub = u1[:dx], u1[dx:]
        r1 = b1.reshape(1, h)
        rc1 = cb1.reshape(1, h)
        pr, ktab = _tc_tables(xc, wr, wc, vx, r1, rc1)
        gr, gk = _sc_gather(pr, ktab, rowg, colg)
        ea, out_e = _tc_edge(gr, gk, ein, we, w2, b2.reshape(1, h),
                             lwe.reshape(1, h), lbe.reshape(1, h),
                             ve, v2, c2.reshape(1, h),
                             lwn.reshape(1, h), lbn.reshape(1, h))
        sums = _sc_scatter(out_e, rows_scat, zeros_h)
        xc = _tc_node(xc, sums, c0, c1cnt, ua, ub, d1.reshape(1, h),
                      lwm.reshape(1, h), lbm.reshape(1, h), u2,
                      d2.reshape(1, h))
        ein = ea

    parents = _sc_parents(xc, cand16)
    w1f, b1f, w2f, b2f, lwf, lbf = mfin
    mo = w2f.shape[1]
    o = _tc_final(parents, w1f, b1f.reshape(1, h), w2f, b2f.reshape(1, mo),
                  lwf.reshape(1, mo), lbf.reshape(1, mo), s_par)
    return o[0]


# speedup vs baseline: 1.4482x; 1.0618x over previous
import functools

import jax
import jax.numpy as jnp
from jax import lax
from jax.experimental import pallas as pl
from jax.experimental.pallas import tpu as pltpu
from jax.experimental.pallas import tpu_sc as plsc

F32 = jnp.float32
NC, NS, LANES = 2, 16, 16
NW = NC * NS
CBG = 64
CBS = 128
BE = 512
BN = 512

@functools.cache
def _mesh():
    return plsc.VectorSubcoreMesh(core_axis_name="c", subcore_axis_name="s",
                                  num_cores=NC, num_subcores=NS)


def _ln(h, lw, lb, eps=1e-5):
    m = jnp.mean(h, axis=-1, keepdims=True)
    v = jnp.mean((h - m) ** 2, axis=-1, keepdims=True)
    return (h - m) * lax.rsqrt(v + eps) * lw + lb


def _tables_body(x_ref, wr_ref, wc_ref, wv_ref, b1_ref, c1_ref, pr_ref, k_ref):
    x = x_ref[...]
    pr_ref[...] = jnp.dot(x, wr_ref[...], preferred_element_type=F32) + b1_ref[...]
    kc = jnp.dot(x, wc_ref[...], preferred_element_type=F32)
    kq = jnp.dot(x, wv_ref[...], preferred_element_type=F32) + c1_ref[...]
    k_ref[...] = jnp.concatenate([kc, kq], axis=1)


def _tc_tables(x, wr, wc, wv, b1, c1):
    np_, dx = x.shape
    h = wr.shape[1]
    grid = (np_ // BN,)
    return pl.pallas_call(
        _tables_body,
        grid=grid,
        in_specs=[
            pl.BlockSpec((BN, dx), lambda i: (i, 0)),
            pl.BlockSpec((dx, h), lambda i: (0, 0)),
            pl.BlockSpec((dx, h), lambda i: (0, 0)),
            pl.BlockSpec((dx, h), lambda i: (0, 0)),
            pl.BlockSpec((1, h), lambda i: (0, 0)),
            pl.BlockSpec((1, h), lambda i: (0, 0)),
        ],
        out_specs=[
            pl.BlockSpec((BN, h), lambda i: (i, 0)),
            pl.BlockSpec((BN, 2 * h), lambda i: (i, 0)),
        ],
        out_shape=[
            jax.ShapeDtypeStruct((np_, h), F32),
            jax.ShapeDtypeStruct((np_, 2 * h), F32),
        ],
    )(x, wr, wc, wv, b1, c1)


def _sc_gather(pr, ktab, row2, col2):
    nchunks, cbg = row2.shape
    ep = nchunks * cbg
    h = pr.shape[1]
    nch = (ep // NW) // cbg

    @functools.partial(
        pl.kernel,
        out_type=[jax.ShapeDtypeStruct((ep, h), F32),
                  jax.ShapeDtypeStruct((ep, 2 * h), F32)],
        mesh=_mesh(),
        scratch_types=[
            pltpu.VMEM((nch, cbg), jnp.int32),
            pltpu.VMEM((nch, cbg), jnp.int32),
            pltpu.VMEM((cbg, h), F32),
            pltpu.VMEM((cbg, 2 * h), F32),
            pltpu.VMEM((cbg, h), F32),
            pltpu.VMEM((cbg, 2 * h), F32),
            pltpu.SemaphoreType.DMA,
            pltpu.SemaphoreType.DMA,
            pltpu.SemaphoreType.DMA,
            pltpu.SemaphoreType.DMA,
        ],
    )
    def k(pr_hbm, k_hbm, row_hbm, col_hbm, gr_hbm, gk_hbm,
          idxr, idxc, bufr_a, bufk_a, bufr_b, bufk_b,
          sem_ra, sem_ka, sem_rb, sem_kb):
        wid = lax.axis_index("s") * NC + lax.axis_index("c")
        cbase = wid * nch
        pltpu.sync_copy(row_hbm.at[pl.ds(cbase, nch)], idxr)
        pltpu.sync_copy(col_hbm.at[pl.ds(cbase, nch)], idxc)
        pltpu.async_copy(pr_hbm.at[idxr.at[0]], bufr_a, sem_ra)
        pltpu.async_copy(k_hbm.at[idxc.at[0]], bufk_a, sem_ka)

        def body(j, carry):
            ca = 2 * j
            pltpu.async_copy(pr_hbm.at[idxr.at[ca + 1]], bufr_b, sem_rb)
            pltpu.async_copy(k_hbm.at[idxc.at[ca + 1]], bufk_b, sem_kb)
            pltpu.make_async_copy(pr_hbm.at[pl.ds(0, cbg)], bufr_a, sem_ra).wait()
            pltpu.sync_copy(bufr_a, gr_hbm.at[pl.ds((cbase + ca) * cbg, cbg)])
            pltpu.make_async_copy(k_hbm.at[pl.ds(0, cbg)], bufk_a, sem_ka).wait()
            pltpu.sync_copy(bufk_a, gk_hbm.at[pl.ds((cbase + ca) * cbg, cbg)])

            @pl.when(ca + 2 < nch)
            def _():
                pltpu.async_copy(pr_hbm.at[idxr.at[ca + 2]], bufr_a, sem_ra)
                pltpu.async_copy(k_hbm.at[idxc.at[ca + 2]], bufk_a, sem_ka)

            pltpu.make_async_copy(pr_hbm.at[pl.ds(0, cbg)], bufr_b, sem_rb).wait()
            pltpu.sync_copy(bufr_b, gr_hbm.at[pl.ds((cbase + ca + 1) * cbg, cbg)])
            pltpu.make_async_copy(k_hbm.at[pl.ds(0, cbg)], bufk_b, sem_kb).wait()
            pltpu.sync_copy(bufk_b, gk_hbm.at[pl.ds((cbase + ca + 1) * cbg, cbg)])
            return carry

        lax.fori_loop(0, nch // 2, body, 0)

    return k(pr, ktab, row2, col2)


def _edge_body(gr_ref, gk_ref, ein_ref, we_ref, w2_ref, b2_ref, lwe_ref,
               lbe_ref, ve_ref, v2_ref, c2_ref, lwn_ref, lbn_ref,
               ea_ref, out_ref):
    h = gr_ref.shape[1]
    pre = (gr_ref[...] + gk_ref[:, :h]
           + jnp.dot(ein_ref[...], we_ref[...], preferred_element_type=F32))
    t = _ln(jnp.maximum(pre, 0.0), lwe_ref[...], lbe_ref[...])
    ea = jnp.dot(t, w2_ref[...], preferred_element_type=F32) + b2_ref[...]
    ea_ref[...] = ea
    pre2 = gk_ref[:, h:] + jnp.dot(ea, ve_ref[...], preferred_element_type=F32)
    u = _ln(jnp.maximum(pre2, 0.0), lwn_ref[...], lbn_ref[...])
    out_ref[...] = jnp.dot(u, v2_ref[...], preferred_element_type=F32) + c2_ref[...]


def _tc_edge(gr, gk, ein, we, w2, b2, lwe, lbe, ve, v2, c2, lwn, lbn):
    ep, h = gr.shape
    de = ein.shape[1]
    grid = (ep // BE,)
    wspec = lambda s: pl.BlockSpec(s, lambda i: (0, 0))
    return pl.pallas_call(
        _edge_body,
        grid=grid,
        in_specs=[
            pl.BlockSpec((BE, h), lambda i: (i, 0)),
            pl.BlockSpec((BE, 2 * h), lambda i: (i, 0)),
            pl.BlockSpec((BE, de), lambda i: (i, 0)),
            wspec((de, h)), wspec((h, h)), wspec((1, h)), wspec((1, h)),
            wspec((1, h)), wspec((h, h)), wspec((h, h)), wspec((1, h)),
            wspec((1, h)), wspec((1, h)),
        ],
        out_specs=[
            pl.BlockSpec((BE, h), lambda i: (i, 0)),
            pl.BlockSpec((BE, h), lambda i: (i, 0)),
        ],
        out_shape=[
            jax.ShapeDtypeStruct((ep, h), F32),
            jax.ShapeDtypeStruct((ep, h), F32),
        ],
    )(gr, gk, ein, we, w2, b2, lwe, lbe, ve, v2, c2, lwn, lbn)


def _sc_scatter(out_e, rows_scat, zeros_h):
    ep = rows_scat.shape[0]
    h = out_e.shape[1]
    hh = h // NC
    eps = ep // NS
    nch = eps // CBS
    np_ = zeros_h.shape[0]
    nps = np_ // NS

    @functools.partial(
        pl.kernel,
        out_type=jax.ShapeDtypeStruct((np_, h), F32),
        mesh=_mesh(),
        scratch_types=[
            pltpu.VMEM((CBS,), jnp.int32),
            pltpu.VMEM((CBS, hh), F32),
            pltpu.VMEM_SHARED((np_, hh), F32),
        ],
    )
    def k(out_hbm, row_hbm, z_hbm, sums_hbm, idxv, datv, acc):
        c = lax.axis_index("c")
        s = lax.axis_index("s")
        pltpu.sync_copy(z_hbm.at[pl.ds(s * nps, nps)],
                        acc.at[pl.ds(s * nps, nps)])
        plsc.subcore_barrier()

        def run(col0):
            def body(j, carry):
                e0 = s * eps + j * CBS
                pltpu.sync_copy(row_hbm.at[pl.ds(e0, CBS)], idxv)
                pltpu.sync_copy(out_hbm.at[pl.ds(e0, CBS), pl.ds(col0, hh)], datv)
                pltpu.sync_copy(datv, acc.at[idxv], add=True)
                return carry
            lax.fori_loop(0, nch, body, 0)

        @pl.when(c == 0)
        def _():
            run(0)

        @pl.when(c == 1)
        def _():
            run(hh)

        plsc.subcore_barrier()

        @pl.when(c == 0)
        def _():
            pltpu.sync_copy(acc.at[pl.ds(s * nps, nps)],
                            sums_hbm.at[pl.ds(s * nps, nps), pl.ds(0, hh)])

        @pl.when(c == 1)
        def _():
            pltpu.sync_copy(acc.at[pl.ds(s * nps, nps)],
                            sums_hbm.at[pl.ds(s * nps, nps), pl.ds(hh, hh)])

    return k(out_e, rows_scat, zeros_h)


def _sc_counts(rows_scat, ones_hbm, zeros_h):
    ep = rows_scat.shape[0]
    epw = ep // NW
    nch = epw // CBS
    np_, cw = zeros_h.shape
    nps = np_ // NS

    @functools.partial(
        pl.kernel,
        out_type=jax.ShapeDtypeStruct((NC * np_, cw), F32),
        mesh=_mesh(),
        scratch_types=[
            pltpu.VMEM((CBS,), jnp.int32),
            pltpu.VMEM((CBS, cw), F32),
            pltpu.VMEM_SHARED((np_, cw), F32),
        ],
    )
    def k(row_hbm, ones_ref, z_hbm, cnt_hbm, idxv, onev, acc):
        c = lax.axis_index("c")
        s = lax.axis_index("s")
        wid = s * NC + c
        pltpu.sync_copy(ones_ref, onev)
        pltpu.sync_copy(z_hbm.at[pl.ds(s * nps, nps)],
                        acc.at[pl.ds(s * nps, nps)])
        plsc.subcore_barrier()

        def body(j, carry):
            e0 = wid * epw + j * CBS
            pltpu.sync_copy(row_hbm.at[pl.ds(e0, CBS)], idxv)
            pltpu.sync_copy(onev, acc.at[idxv], add=True)
            return carry

        lax.fori_loop(0, nch, body, 0)
        plsc.subcore_barrier()

        @pl.when(c == 0)
        def _():
            pltpu.sync_copy(acc.at[pl.ds(s * nps, nps)],
                            cnt_hbm.at[pl.ds(s * nps, nps)])

        @pl.when(c == 1)
        def _():
            pltpu.sync_copy(acc.at[pl.ds(s * nps, nps)],
                            cnt_hbm.at[pl.ds(np_ + s * nps, nps)])

    return k(rows_scat, ones_hbm, zeros_h)


def _node_body(x_ref, sums_ref, c0_ref, c1_ref, ua_ref, ub_ref, d1_ref,
               lw_ref, lb_ref, u2_ref, d2_ref, xo_ref):
    cnt = c0_ref[:, 0:1] + c1_ref[:, 0:1]
    inv = 1.0 / jnp.maximum(cnt, 1.0)
    agg = sums_ref[...] * inv
    pre = (jnp.dot(x_ref[...], ua_ref[...], preferred_element_type=F32)
           + jnp.dot(agg, ub_ref[...], preferred_element_type=F32) + d1_ref[...])
    hh = _ln(jnp.maximum(pre, 0.0), lw_ref[...], lb_ref[...])
    xo_ref[...] = jnp.dot(hh, u2_ref[...], preferred_element_type=F32) + d2_ref[...]


def _tc_node(x, sums, c0, c1, ua, ub, d1, lw, lb, u2, d2):
    np_, dx = x.shape
    h = sums.shape[1]
    grid = (np_ // BN,)
    wspec = lambda s: pl.BlockSpec(s, lambda i: (0, 0))
    return pl.pallas_call(
        _node_body,
        grid=grid,
        in_specs=[
            pl.BlockSpec((BN, dx), lambda i: (i, 0)),
            pl.BlockSpec((BN, h), lambda i: (i, 0)),
            pl.BlockSpec((BN, c0.shape[1]), lambda i: (i, 0)),
            pl.BlockSpec((BN, c1.shape[1]), lambda i: (i, 0)),
            wspec((dx, h)), wspec((h, h)), wspec((1, h)), wspec((1, h)),
            wspec((1, h)), wspec((h, h)), wspec((1, h)),
        ],
        out_specs=pl.BlockSpec((BN, h), lambda i: (i, 0)),
        out_shape=jax.ShapeDtypeStruct((np_, h), F32),
    )(x, sums, c0, c1, ua, ub, d1, lw, lb, u2, d2)


def _sc_parents(x, cand16):
    h = x.shape[1]

    @functools.partial(
        pl.kernel,
        out_type=jax.ShapeDtypeStruct((LANES, h), F32),
        mesh=_mesh(),
        scratch_types=[
            pltpu.VMEM((LANES,), jnp.int32),
            pltpu.VMEM((LANES, h), F32),
            pltpu.SemaphoreType.DMA,
        ],
    )
    def k(x_hbm, cand_hbm, par_hbm, idxv, bufv, sem):
        c = lax.axis_index("c")
        s = lax.axis_index("s")

        @pl.when(jnp.logical_and(c == 0, s == 0))
        def _():
            pltpu.sync_copy(cand_hbm, idxv)
            pltpu.async_copy(x_hbm.at[idxv], bufv, sem).wait()
            pltpu.sync_copy(bufv, par_hbm)

    return k(x, cand16)


def _final_body(par_ref, w1_ref, b1_ref, w2_ref, b2_ref, lw_ref, lb_ref,
                o_ref, *, n_par):
    h = par_ref.shape[1]
    p = par_ref[...]
    acc = jnp.zeros((1, h), F32)
    for j in range(n_par):
        acc = acc + jnp.dot(p[j:j + 1, :], w1_ref[j * h:(j + 1) * h, :],
                            preferred_element_type=F32)
    hid = jnp.maximum(acc + b1_ref[...], 0.0)
    o = jnp.dot(hid, w2_ref[...], preferred_element_type=F32) + b2_ref[...]
    o = _ln(o, lw_ref[...], lb_ref[...])
    o = 1.0 / (1.0 + jnp.exp(-o))
    o_ref[...] = jnp.broadcast_to(o, o_ref.shape)


def _tc_final(parents, w1, b1, w2, b2, lw, lb, n_par):
    h = parents.shape[1]
    mo = w2.shape[1]
    return pl.pallas_call(
        functools.partial(_final_body, n_par=n_par),
        out_shape=jax.ShapeDtypeStruct((8, mo), F32),
    )(parents, w1, b1, w2, b2, lw, lb)


def _pad_rows(a, n):
    return jnp.pad(a, ((0, n - a.shape[0]),) + ((0, 0),) * (a.ndim - 1))


def kernel(x, edge_index, edge_attr, candidate, e1, n1a, n1b, e2, n2a, n2b,
           e3, n3a, n3b, mfin):
    n, df = x.shape
    e = edge_index.shape[1]
    h = e1[0].shape[1]
    s_par = candidate.shape[0] - 1

    np_ = ((n + BN - 1) // BN) * BN
    align = NW * CBS
    ep = ((e + align - 1) // align) * align

    row = edge_index[0].astype(jnp.int32)
    col = edge_index[1].astype(jnp.int32)
    rowg = jnp.pad(row, (0, ep - e))
    colg = jnp.pad(col, (0, ep - e))
    rows_scat = jnp.pad(row, (0, ep - e), constant_values=np_ - 1)
    ein = _pad_rows(edge_attr.astype(F32), ep)
    xc = _pad_rows(x.astype(F32), np_)
    zeros_h = jnp.zeros((np_, h // NC), F32)
    ones_blk = jnp.ones((CBS, h // NC), F32)
    cand16 = jnp.zeros((LANES,), jnp.int32).at[:s_par].set(
        candidate[1:1 + s_par].astype(jnp.int32))

    cnt = _sc_counts(rows_scat, ones_blk, zeros_h)
    c0, c1cnt = cnt[:np_], cnt[np_:]

    for pe, pn1, pn2 in ((e1, n1a, n1b), (e2, n2a, n2b), (e3, n3a, n3b)):
        w1, b1, lwe, lbe, w2, b2 = pe
        v1, cb1, lwn, lbn, v2, c2 = pn1
        u1, d1, lwm, lbm, u2, d2 = pn2
        dx = xc.shape[1]
        wr, wc, we = w1[:dx], w1[dx:2 * dx], w1[2 * dx:]
        vx, ve = v1[:dx], v1[dx:]
        ua, ub = u1[:dx], u1[dx:]
        r1 = b1.reshape(1, h)
        rc1 = cb1.reshape(1, h)
        pr, ktab = _tc_tables(xc, wr, wc, vx, r1, rc1)
        gr, gk = _sc_gather(pr, ktab, rowg.reshape(-1, CBG),
                            colg.reshape(-1, CBG))
        ea, out_e = _tc_edge(gr, gk, ein, we, w2, b2.reshape(1, h),
                             lwe.reshape(1, h), lbe.reshape(1, h),
                             ve, v2, c2.reshape(1, h),
                             lwn.reshape(1, h), lbn.reshape(1, h))
        sums = _sc_scatter(out_e, rows_scat, zeros_h)
        xc = _tc_node(xc, sums, c0, c1cnt, ua, ub, d1.reshape(1, h),
                      lwm.reshape(1, h), lbm.reshape(1, h), u2,
                      d2.reshape(1, h))
        ein = ea

    parents = _sc_parents(xc, cand16)
    w1f, b1f, w2f, b2f, lwf, lbf = mfin
    mo = w2f.shape[1]
    o = _tc_final(parents, w1f, b1f.reshape(1, h), w2f, b2f.reshape(1, mo),
                  lwf.reshape(1, mo), lbf.reshape(1, mo), s_par)
    return o[0]
